# baseline (speedup 1.0000x reference)
import functools

import jax
import jax.numpy as jnp
from jax import lax
from jax.experimental import pallas as pl
from jax.experimental.pallas import tpu as pltpu
from jax.experimental.pallas import tpu_sc as plsc

BATCH = 1024
INPUTS = 4096
CLASSES = 10
FIN = 16
ENTRIES = 8192
HENT = ENTRIES // 2
HFN = 2
FPD = INPUTS // FIN

NC, NS, L = 2, 16, 16
NW = NC * NS
ROWS_PER_W = BATCH // NW
F_PER_W = FPD // NW


_PACK_BF = 32


def _pack_body(t_ref, o_any, scratch, sem):
    i = pl.program_id(0)
    lo = jnp.zeros((_PACK_BF, HENT), jnp.int32)
    hi = jnp.zeros((_PACK_BF, HENT), jnp.int32)
    for c in range(CLASSES):
        tc = t_ref[c]
        lo = lo | jnp.where(tc[:, :HENT] >= 0.0, jnp.int32(1 << c),
                            jnp.int32(0))
        hi = hi | jnp.where(tc[:, HENT:] >= 0.0, jnp.int32(1 << c),
                            jnp.int32(0))
    scratch[...] = lo | (hi << 16)
    copies = []
    for k in range(_PACK_BF):
        cp = pltpu.make_async_copy(
            scratch.at[k],
            o_any.at[pl.ds((i * _PACK_BF + k) * HENT, HENT)],
            sem)
        cp.start()
        copies.append(cp)
    for cp in copies:
        cp.wait()


def _pack_table(table):
    return pl.pallas_call(
        _pack_body,
        grid=(FPD // _PACK_BF,),
        in_specs=[pl.BlockSpec((CLASSES, _PACK_BF, ENTRIES),
                               lambda i: (0, i, 0))],
        out_specs=pl.BlockSpec(memory_space=pl.ANY),
        out_shape=jax.ShapeDtypeStruct((FPD * HENT,), jnp.int32),
        scratch_shapes=[pltpu.VMEM((_PACK_BF, HENT), jnp.int32),
                        pltpu.SemaphoreType.DMA],
    )(table)


def _hash_body(x_hbm, gidx_hbm, hvb_hbm, idx_hbm,
               xp_v, gidx_v, hvb_v, idxbuf_v, sem_x):
    wid = lax.axis_index("s") * NC + lax.axis_index("c")
    base = wid * ROWS_PER_W

    pltpu.sync_copy(gidx_hbm, gidx_v)
    pltpu.sync_copy(hvb_hbm, hvb_v)

    hv0 = [hvb_v[pl.ds(i * L, L)] for i in range(FIN)]
    hv1 = [hvb_v[pl.ds(FPD + i * L, L)] for i in range(FIN)]

    lanes = lax.iota(jnp.int32, L)
    kbase = lanes >> 3
    pbase = lanes & 7
    zvec = jnp.zeros((L,), jnp.int32)
    R = 4
    NCH = ROWS_PER_W // R

    def hash_chunk(rowsel, c):
        for g in range(16):
            acc0 = [jnp.zeros((L,), jnp.int32) for _ in range(R)]
            acc1 = [jnp.zeros((L,), jnp.int32) for _ in range(R)]
            for i in range(FIN):
                gv = gidx_v[pl.ds(g * 256 + i * L, L)]
                for r in range(R):
                    v = plsc.load_gather(xp_v, [rowsel[r], gv])
                    acc0[r] = acc0[r] ^ (v * hv0[i])
                    acc1[r] = acc1[r] ^ (v * hv1[i])
            kvec = kbase + (2 * g)
            for r in range(R):
                rowpos = (c * R + r) * L
                plsc.store_scatter(idxbuf_v, [kvec, zvec, pbase + rowpos],
                                   acc0[r])
                plsc.store_scatter(idxbuf_v,
                                   [kvec, zvec, pbase + (rowpos + 8)],
                                   acc1[r])

    pltpu.async_copy(x_hbm.at[pl.ds(base, R)], xp_v.at[pl.ds(0, R)], sem_x)

    def p1_body(c, _):
        par = (c & 1) * R
        a = base + c * R
        pltpu.make_async_copy(x_hbm.at[pl.ds(a, R)],
                              xp_v.at[pl.ds(par, R)], sem_x).wait()

        @pl.when(c < NCH - 1)
        def _():
            pltpu.async_copy(x_hbm.at[pl.ds(a + R, R)],
                             xp_v.at[pl.ds(R - par, R)], sem_x)

        rowsel = [jnp.full((L,), par + r, jnp.int32) for r in range(R)]
        hash_chunk(rowsel, c)
        return ()

    lax.fori_loop(0, NCH, p1_body, ())
    pltpu.sync_copy(idxbuf_v,
                    idx_hbm.at[:, pl.ds(wid // 4, 1),
                               pl.ds((wid % 4) * 512, 512)])


def _hash_run(x_b, gidx, hvb):
    mesh = plsc.VectorSubcoreMesh(core_axis_name="c", subcore_axis_name="s")
    f = pl.kernel(
        _hash_body,
        out_type=jax.ShapeDtypeStruct((NW, 8, 2048), jnp.int32),
        mesh=mesh,
        compiler_params=pltpu.CompilerParams(needs_layout_passes=False),
        scratch_types=[
            pltpu.VMEM((8, INPUTS), jnp.int32),
            pltpu.VMEM((INPUTS,), jnp.int32),
            pltpu.VMEM((HFN * FPD,), jnp.int32),
            pltpu.VMEM((NW, 1, 512), jnp.int32),
            pltpu.SemaphoreType.DMA,
        ],
    )
    return f(x_b, gidx, hvb)


def _combine_body(packed_hbm, idx_hbm, part_hbm,
                  slab_v, ih_v, part_v, sem):
    wid = lax.axis_index("s") * NC + lax.axis_index("c")
    fbase = wid * F_PER_W

    cp_s = pltpu.make_async_copy(
        packed_hbm.at[pl.ds(fbase * HENT, F_PER_W * HENT)], slab_v, sem)
    cp_s.start()
    cp_i = pltpu.make_async_copy(
        idx_hbm.at[pl.ds(wid * BATCH * 2 * F_PER_W, BATCH * 2 * F_PER_W)],
        ih_v, sem)
    cp_i.start()
    cp_s.wait()
    cp_i.wait()

    lanes = lax.iota(jnp.int32, L)
    pairmask = [jnp.full((L,), (1 << k) | (1 << (k + 5)), jnp.int32)
                for k in range(5)]

    def bc_body(bc, _):
        bv16 = (lanes + bc * L) * (2 * F_PER_W)
        accp = [jnp.zeros((L,), jnp.int32) for _ in range(5)]
        for f in range(F_PER_W):
            e0 = plsc.load_gather(ih_v, [bv16 + f])
            e1 = plsc.load_gather(ih_v, [bv16 + (F_PER_W + f)])
            w0 = plsc.load_gather(slab_v, [(e0 & (HENT - 1)) + f * HENT])
            w1 = plsc.load_gather(slab_v, [(e1 & (HENT - 1)) + f * HENT])
            m0 = (w0 >> ((e0 & HENT) >> 8)) & 1023
            m1 = (w1 >> ((e1 & HENT) >> 8)) & 1023
            m = m0 & m1
            for k in range(5):
                accp[k] = accp[k] + (m & pairmask[k])
        for k in range(5):
            part_v[pl.ds(k * BATCH + bc * L, L)] = (accp[k] >> k) & 15
            part_v[pl.ds((k + 5) * BATCH + bc * L, L)] = \
                (accp[k] >> (k + 5)) & 15
        return ()

    lax.fori_loop(0, BATCH // L, bc_body, ())
    pltpu.sync_copy(part_v, part_hbm.at[pl.ds(wid * 16 * BATCH, 16 * BATCH)])


def _combine_run(packed_flat, idx2):
    mesh = plsc.VectorSubcoreMesh(core_axis_name="c", subcore_axis_name="s")
    f = pl.kernel(
        _combine_body,
        out_type=jax.ShapeDtypeStruct((NW * 16 * BATCH,), jnp.int32),
        mesh=mesh,
        compiler_params=pltpu.CompilerParams(needs_layout_passes=False),
        scratch_types=[
            pltpu.VMEM((F_PER_W * HENT,), jnp.int32),
            pltpu.VMEM((BATCH * 2 * F_PER_W,), jnp.int32),
            pltpu.VMEM((16 * BATCH,), jnp.int32),
            pltpu.SemaphoreType.DMA,
        ],
    )
    return f(packed_flat, idx2)


def _reduce_body(p_ref, b_ref, o_ref):
    acc = jnp.zeros((16, BATCH), jnp.int32)
    for t in range(NW):
        acc = acc + p_ref[t]
    o_ref[...] = (2.0 * acc.astype(jnp.float32) - jnp.float32(FPD)
                  + b_ref[...])


def _reduce_run(part, bias_col):
    return pl.pallas_call(
        _reduce_body,
        out_shape=jax.ShapeDtypeStruct((16, BATCH), jnp.float32),
    )(part, bias_col)


@functools.partial(jax.jit, static_argnames=())
def kernel(x_b, table, hash_values, input_order, bias):
    gidx = input_order.reshape(16, 16, 16).transpose(0, 2, 1).reshape(-1)
    hvb = jnp.broadcast_to(hash_values[:, :, None], (HFN, FIN, L)).reshape(-1)
    idx = _hash_run(x_b, gidx, hvb)
    packed_flat = _pack_table(table)
    part = _combine_run(packed_flat, idx.reshape(-1))
    bias_col = jnp.pad(bias, (0, 16 - CLASSES))[:, None]
    acts = _reduce_run(part.reshape(NW, 16, BATCH),
                       jnp.broadcast_to(bias_col, (16, BATCH)))
    return acts.T[:, :CLASSES]


# speedup vs baseline: 1.0500x; 1.0500x over previous
import functools

import jax
import jax.numpy as jnp
from jax import lax
from jax.experimental import pallas as pl
from jax.experimental.pallas import tpu as pltpu
from jax.experimental.pallas import tpu_sc as plsc

BATCH = 1024
INPUTS = 4096
CLASSES = 10
FIN = 16
ENTRIES = 8192
HENT = ENTRIES // 2
HFN = 2
FPD = INPUTS // FIN

NC, NS, L = 2, 16, 16
NW = NC * NS
ROWS_PER_W = BATCH // NW
F_PER_W = FPD // NW


_PACK_BF = 32


def _pack_body(t_ref, o_any, scratch, sem):
    i = pl.program_id(0)
    lo = jnp.zeros((_PACK_BF, HENT), jnp.int32)
    hi = jnp.zeros((_PACK_BF, HENT), jnp.int32)
    for c in range(CLASSES):
        tc = t_ref[c]
        lo = lo | jnp.where(tc[:, :HENT] >= 0.0, jnp.int32(1 << c),
                            jnp.int32(0))
        hi = hi | jnp.where(tc[:, HENT:] >= 0.0, jnp.int32(1 << c),
                            jnp.int32(0))
    scratch[...] = lo | (hi << 16)
    copies = []
    for k in range(_PACK_BF):
        cp = pltpu.make_async_copy(
            scratch.at[k],
            o_any.at[pl.ds((i * _PACK_BF + k) * HENT, HENT)],
            sem)
        cp.start()
        copies.append(cp)
    for cp in copies:
        cp.wait()


def _pack_table(table):
    return pl.pallas_call(
        _pack_body,
        grid=(FPD // _PACK_BF,),
        in_specs=[pl.BlockSpec((CLASSES, _PACK_BF, ENTRIES),
                               lambda i: (0, i, 0))],
        out_specs=pl.BlockSpec(memory_space=pl.ANY),
        out_shape=jax.ShapeDtypeStruct((FPD * HENT,), jnp.int32),
        scratch_shapes=[pltpu.VMEM((_PACK_BF, HENT), jnp.int32),
                        pltpu.SemaphoreType.DMA],
    )(table)


def _hash_body(x_hbm, gidx_hbm, hvb_hbm, idx_hbm,
               xp_v, gidx_v, hvb_v, idxbuf_v, sem_x):
    wid = lax.axis_index("s") * NC + lax.axis_index("c")
    base = wid * ROWS_PER_W

    pltpu.sync_copy(gidx_hbm, gidx_v)
    pltpu.sync_copy(hvb_hbm, hvb_v)

    hv0 = [hvb_v[pl.ds(i * L, L)] for i in range(FIN)]
    hv1 = [hvb_v[pl.ds(FPD + i * L, L)] for i in range(FIN)]

    lanes = lax.iota(jnp.int32, L)
    kbase = lanes >> 3
    pbase = lanes & 7
    R = 4
    NCH = ROWS_PER_W // R

    def hash_chunk(rowsel, c):
        for g in range(16):
            acc0 = [jnp.zeros((L,), jnp.int32) for _ in range(R)]
            acc1 = [jnp.zeros((L,), jnp.int32) for _ in range(R)]
            for i in range(FIN):
                gv = gidx_v[pl.ds(g * 256 + i * L, L)]
                for r in range(R):
                    v = plsc.load_gather(xp_v, [rowsel[r], gv])
                    acc0[r] = acc0[r] ^ (v * hv0[i])
                    acc1[r] = acc1[r] ^ (v * hv1[i])
            kvec = kbase + (2 * g)
            for r in range(R):
                rowpos = (c * R + r) * L
                plsc.store_scatter(idxbuf_v, [kvec, pbase + rowpos], acc0[r])
                plsc.store_scatter(idxbuf_v, [kvec, pbase + (rowpos + 8)],
                                   acc1[r])

    pltpu.async_copy(x_hbm.at[pl.ds(base, R)], xp_v.at[pl.ds(0, R)], sem_x)

    def p1_body(c, _):
        par = (c & 1) * R
        a = base + c * R
        pltpu.make_async_copy(x_hbm.at[pl.ds(a, R)],
                              xp_v.at[pl.ds(par, R)], sem_x).wait()

        @pl.when(c < NCH - 1)
        def _():
            pltpu.async_copy(x_hbm.at[pl.ds(a + R, R)],
                             xp_v.at[pl.ds(R - par, R)], sem_x)

        rowsel = [jnp.full((L,), par + r, jnp.int32) for r in range(R)]
        hash_chunk(rowsel, c)
        return ()

    lax.fori_loop(0, NCH, p1_body, ())
    pltpu.sync_copy(idxbuf_v, idx_hbm.at[:, pl.ds(wid * 512, 512)])


def _hash_run(x_b, gidx, hvb):
    mesh = plsc.VectorSubcoreMesh(core_axis_name="c", subcore_axis_name="s")
    f = pl.kernel(
        _hash_body,
        out_type=jax.ShapeDtypeStruct((NW, ROWS_PER_W * 512), jnp.int32),
        mesh=mesh,
        compiler_params=pltpu.CompilerParams(needs_layout_passes=False),
        scratch_types=[
            pltpu.VMEM((8, INPUTS), jnp.int32),
            pltpu.VMEM((INPUTS,), jnp.int32),
            pltpu.VMEM((HFN * FPD,), jnp.int32),
            pltpu.VMEM((NW, 512), jnp.int32),
            pltpu.SemaphoreType.DMA,
        ],
    )
    return f(x_b, gidx, hvb)


def _combine_body(packed_hbm, idx_hbm, part_hbm,
                  slab_v, ih_v, part_v, sem):
    wid = lax.axis_index("s") * NC + lax.axis_index("c")
    fbase = wid * F_PER_W

    cp_s = pltpu.make_async_copy(
        packed_hbm.at[pl.ds(fbase * HENT, F_PER_W * HENT)], slab_v, sem)
    cp_s.start()
    cp_i = pltpu.make_async_copy(idx_hbm.at[wid], ih_v, sem)
    cp_i.start()
    cp_s.wait()
    cp_i.wait()

    lanes = lax.iota(jnp.int32, L)
    pairmask = [jnp.full((L,), (1 << k) | (1 << (k + 5)), jnp.int32)
                for k in range(5)]

    def bc_body(bc, _):
        bv16 = (lanes + bc * L) * (2 * F_PER_W)
        accp = [jnp.zeros((L,), jnp.int32) for _ in range(5)]
        for f in range(F_PER_W):
            e0 = plsc.load_gather(ih_v, [bv16 + f])
            e1 = plsc.load_gather(ih_v, [bv16 + (F_PER_W + f)])
            w0 = plsc.load_gather(slab_v, [(e0 & (HENT - 1)) + f * HENT])
            w1 = plsc.load_gather(slab_v, [(e1 & (HENT - 1)) + f * HENT])
            m0 = (w0 >> ((e0 & HENT) >> 8)) & 1023
            m1 = (w1 >> ((e1 & HENT) >> 8)) & 1023
            m = m0 & m1
            for k in range(5):
                accp[k] = accp[k] + (m & pairmask[k])
        for k in range(5):
            part_v[k, pl.ds(bc * L, L)] = (accp[k] >> k) & 15
            part_v[k + 5, pl.ds(bc * L, L)] = (accp[k] >> (k + 5)) & 15
        return ()

    lax.fori_loop(0, BATCH // L, bc_body, ())
    pltpu.sync_copy(part_v, part_hbm.at[wid])


def _combine_run(packed_flat, idx2):
    mesh = plsc.VectorSubcoreMesh(core_axis_name="c", subcore_axis_name="s")
    f = pl.kernel(
        _combine_body,
        out_type=jax.ShapeDtypeStruct((NW, 16, BATCH), jnp.int32),
        mesh=mesh,
        compiler_params=pltpu.CompilerParams(needs_layout_passes=False),
        scratch_types=[
            pltpu.VMEM((F_PER_W * HENT,), jnp.int32),
            pltpu.VMEM((BATCH * 2 * F_PER_W,), jnp.int32),
            pltpu.VMEM((16, BATCH), jnp.int32),
            pltpu.SemaphoreType.DMA,
        ],
    )
    return f(packed_flat, idx2)


def _reduce_body(p_ref, b_ref, o_ref):
    acc = jnp.zeros((16, BATCH), jnp.int32)
    for t in range(NW):
        acc = acc + p_ref[t]
    o_ref[...] = (2.0 * acc.astype(jnp.float32) - jnp.float32(FPD)
                  + jnp.broadcast_to(b_ref[...], (16, BATCH)))


def _reduce_run(part, bias_col):
    return pl.pallas_call(
        _reduce_body,
        out_shape=jax.ShapeDtypeStruct((16, BATCH), jnp.float32),
    )(part, bias_col)


@functools.partial(jax.jit, static_argnames=())
def kernel(x_b, table, hash_values, input_order, bias):
    gidx = input_order.reshape(16, 16, 16).transpose(0, 2, 1).reshape(-1)
    hvb = jnp.broadcast_to(hash_values[:, :, None], (HFN, FIN, L)).reshape(-1)
    idx = _hash_run(x_b, gidx, hvb)
    packed_flat = _pack_table(table)
    part = _combine_run(packed_flat, idx)
    bias_col = jnp.pad(bias, (0, 16 - CLASSES))[:, None]
    acts = _reduce_run(part, bias_col)
    return acts.T[:, :CLASSES]
